# speedup vs baseline: 1.1937x; 1.1937x over previous
import functools

import jax
import jax.numpy as jnp
from jax import lax
from jax.experimental import pallas as pl
from jax.experimental.pallas import tpu as pltpu
from jax.experimental.pallas import tpu_sc as plsc

L = 16
NC = 2
NS = 16
NW = NC * NS

CHUNK = 64


def _make_sc_kernel(B, D, n_chunks):
    mesh = plsc.VectorSubcoreMesh(core_axis_name="c", subcore_axis_name="s")

    @functools.partial(
        pl.kernel,
        out_type=jax.ShapeDtypeStruct((B, D), jnp.float32),
        mesh=mesh,
        scratch_types=[
            pltpu.VMEM((2 * CHUNK,), jnp.int32),
            pltpu.VMEM((2 * CHUNK, D), jnp.float32),
            pltpu.VMEM((CHUNK, D), jnp.float32),
            pltpu.VMEM((2, D), jnp.float32),
            pltpu.SemaphoreType.DMA,
        ],
    )
    def decoder_kernel(idx_hbm, table_hbm, w_hbm, out_hbm,
                       idx_v, rows_v, out_v, w_v, sem):
        wid = lax.axis_index("s") * NC + lax.axis_index("c")
        pltpu.sync_copy(w_hbm, w_v)

        def chunk_body(g, carry):
            base = (wid * n_chunks + g) * CHUNK
            pltpu.sync_copy(idx_hbm.at[pl.ds(2 * base, 2 * CHUNK)], idx_v)
            pltpu.async_copy(table_hbm.at[idx_v], rows_v, sem).wait()

            for dj in range(D // L):
                dsl = pl.ds(dj * L, L)
                w0 = w_v[0, dsl]
                w1 = w_v[1, dsl]

                def c_body(c, _):
                    r0 = rows_v[2 * c, dsl]
                    r1 = rows_v[2 * c + 1, dsl]
                    out_v[c, dsl] = jnp.maximum(r0 * w0 + r1 * w1, 0.0)
                    return 0

                lax.fori_loop(0, CHUNK, c_body, 0, unroll=4)

            pltpu.sync_copy(out_v, out_hbm.at[pl.ds(base, CHUNK), :])
            return carry

        lax.fori_loop(0, n_chunks, chunk_body, 0)

    return decoder_kernel


@jax.jit
def kernel(y, table, emb_scale, conv_w, conv_scale):
    B, ctx = y.shape
    V, D = table.shape
    w = (conv_w * jnp.exp(emb_scale + conv_scale)).astype(jnp.float32)
    w_t = w.T
    idx = y.reshape(-1).astype(jnp.int32)

    n_chunks = B // (NW * CHUNK)
    out = _make_sc_kernel(B, D, n_chunks)(idx, table, w_t)
    return out.reshape(B, 1, D)


# speedup vs baseline: 4.3032x; 3.6050x over previous
import functools

import jax
import jax.numpy as jnp
from jax import lax
from jax.experimental import pallas as pl
from jax.experimental.pallas import tpu as pltpu
from jax.experimental.pallas import tpu_sc as plsc

L = 16
NC = 2
NS = 16
NW = NC * NS

CHUNK = 32
GROUP = 8
BLK = 128


def _make_sc_kernel(B, D, n_chunks):
    mesh = plsc.VectorSubcoreMesh(core_axis_name="c", subcore_axis_name="s")
    n_pairs = n_chunks // 2
    idx_per_w = n_chunks * 2 * CHUNK

    @functools.partial(
        pl.kernel,
        out_type=jax.ShapeDtypeStruct((B, 1, D), jnp.float32),
        mesh=mesh,
        scratch_types=[
            pltpu.VMEM((idx_per_w,), jnp.int32),
            pltpu.VMEM((CHUNK, D), jnp.float32),
            pltpu.VMEM((CHUNK, D), jnp.float32),
            pltpu.VMEM((CHUNK, D), jnp.float32),
            pltpu.VMEM((CHUNK, D), jnp.float32),
            pltpu.VMEM((CHUNK, D), jnp.float32),
            pltpu.VMEM((CHUNK, D), jnp.float32),
            pltpu.VMEM((2, D), jnp.float32),
            pltpu.SemaphoreType.DMA,
            pltpu.SemaphoreType.DMA,
            pltpu.SemaphoreType.DMA,
            pltpu.SemaphoreType.DMA,
        ],
    )
    def decoder_kernel(idx_hbm, table_hbm, w_hbm, out_hbm,
                       idx_all, rowsa0, rowsb0, rowsa1, rowsb1,
                       outb0, outb1, w_v,
                       gsem0, gsem1, osem0, osem1):
        wid = lax.axis_index("s") * NC + lax.axis_index("c")
        pltpu.sync_copy(w_hbm, w_v)
        pltpu.sync_copy(idx_hbm.at[pl.ds(wid * idx_per_w, idx_per_w)], idx_all)

        def off0(g):
            return (g // 4) * (2 * BLK) + (g % 4) * CHUNK

        def fire(g, rowsa, rowsb, gsem):
            o0 = off0(g)
            pltpu.async_copy(table_hbm.at[idx_all.at[pl.ds(o0, CHUNK)]],
                             rowsa, gsem)
            pltpu.async_copy(table_hbm.at[idx_all.at[pl.ds(o0 + BLK, CHUNK)]],
                             rowsb, gsem)

        def drain(g, rowsa, rowsb, gsem):
            o0 = off0(g)
            pltpu.make_async_copy(table_hbm.at[idx_all.at[pl.ds(o0, CHUNK)]],
                                  rowsa, gsem).wait()
            pltpu.make_async_copy(
                table_hbm.at[idx_all.at[pl.ds(o0 + BLK, CHUNK)]],
                rowsb, gsem).wait()

        fire(0, rowsa0, rowsb0, gsem0)

        def compute(rowsa, rowsb, outb):
            for grp in range(D // (L * GROUP)):
                wslices = []
                for j in range(GROUP):
                    dsl = pl.ds((grp * GROUP + j) * L, L)
                    wslices.append((w_v[0, dsl], w_v[1, dsl], dsl))

                @plsc.parallel_loop(0, CHUNK, unroll=4)
                def _(c):
                    for w0, w1, dsl in wslices:
                        r0 = rowsa[c, dsl]
                        r1 = rowsb[c, dsl]
                        outb[c, dsl] = jnp.maximum(r0 * w0 + r1 * w1, 0.0)

        def pair_body(i, carry):
            g0 = 2 * i
            g1 = g0 + 1
            base0 = (wid * n_chunks + g0) * CHUNK
            base1 = base0 + CHUNK
            fire(g1, rowsa1, rowsb1, gsem1)
            drain(g0, rowsa0, rowsb0, gsem0)

            @pl.when(i > 0)
            def _():
                pltpu.make_async_copy(
                    outb0, out_hbm.at[pl.ds(base0, CHUNK), 0], osem0).wait()

            compute(rowsa0, rowsb0, outb0)
            pltpu.async_copy(outb0, out_hbm.at[pl.ds(base0, CHUNK), 0], osem0)

            @pl.when(i < n_pairs - 1)
            def _():
                fire(g0 + 2, rowsa0, rowsb0, gsem0)

            drain(g1, rowsa1, rowsb1, gsem1)

            @pl.when(i > 0)
            def _():
                pltpu.make_async_copy(
                    outb1, out_hbm.at[pl.ds(base1, CHUNK), 0], osem1).wait()

            compute(rowsa1, rowsb1, outb1)
            pltpu.async_copy(outb1, out_hbm.at[pl.ds(base1, CHUNK), 0], osem1)
            return carry

        lax.fori_loop(0, n_pairs, pair_body, 0)
        pltpu.make_async_copy(outb0, out_hbm.at[pl.ds(0, CHUNK), 0], osem0).wait()
        pltpu.make_async_copy(outb1, out_hbm.at[pl.ds(0, CHUNK), 0], osem1).wait()

    return decoder_kernel


@jax.jit
def kernel(y, table, emb_scale, conv_w, conv_scale):
    B, ctx = y.shape
    V, D = table.shape
    w = (conv_w * jnp.exp(emb_scale + conv_scale)).astype(jnp.float32)
    w_t = w.T
    n_chunks = B // (NW * CHUNK)
    idx = (y.astype(jnp.int32)
            .reshape(B // BLK, BLK, ctx)
            .transpose(0, 2, 1)
            .reshape(-1))

    return _make_sc_kernel(B, D, n_chunks)(idx, table, w_t)


# speedup vs baseline: 4.3070x; 1.0009x over previous
import functools

import jax
import jax.numpy as jnp
from jax import lax
from jax.experimental import pallas as pl
from jax.experimental.pallas import tpu as pltpu
from jax.experimental.pallas import tpu_sc as plsc

L = 16
NC = 2
NS = 16
NW = NC * NS

CHUNK = 32
GROUP = 16
BLK = 128


def _make_sc_kernel(B, D, n_chunks):
    mesh = plsc.VectorSubcoreMesh(core_axis_name="c", subcore_axis_name="s")
    n_pairs = n_chunks // 2
    idx_per_w = n_chunks * 2 * CHUNK

    @functools.partial(
        pl.kernel,
        out_type=jax.ShapeDtypeStruct((B, 1, D), jnp.float32),
        mesh=mesh,
        scratch_types=[
            pltpu.VMEM((idx_per_w,), jnp.int32),
            pltpu.VMEM((CHUNK, D), jnp.float32),
            pltpu.VMEM((CHUNK, D), jnp.float32),
            pltpu.VMEM((CHUNK, D), jnp.float32),
            pltpu.VMEM((CHUNK, D), jnp.float32),
            pltpu.VMEM((CHUNK, D), jnp.float32),
            pltpu.VMEM((CHUNK, D), jnp.float32),
            pltpu.VMEM((2, D), jnp.float32),
            pltpu.SemaphoreType.DMA,
            pltpu.SemaphoreType.DMA,
            pltpu.SemaphoreType.DMA,
            pltpu.SemaphoreType.DMA,
        ],
    )
    def decoder_kernel(idx_hbm, table_hbm, w_hbm, out_hbm,
                       idx_all, rowsa0, rowsb0, rowsa1, rowsb1,
                       outb0, outb1, w_v,
                       gsem0, gsem1, osem0, osem1):
        wid = lax.axis_index("s") * NC + lax.axis_index("c")
        pltpu.sync_copy(w_hbm, w_v)
        pltpu.sync_copy(idx_hbm.at[pl.ds(wid * idx_per_w, idx_per_w)], idx_all)

        def off0(g):
            cpb = BLK // CHUNK
            return (g // cpb) * (2 * BLK) + (g % cpb) * CHUNK

        def fire(g, rowsa, rowsb, gsem):
            o0 = off0(g)
            pltpu.async_copy(table_hbm.at[idx_all.at[pl.ds(o0, CHUNK)]],
                             rowsa, gsem)
            pltpu.async_copy(table_hbm.at[idx_all.at[pl.ds(o0 + BLK, CHUNK)]],
                             rowsb, gsem)

        def drain(g, rowsa, rowsb, gsem):
            o0 = off0(g)
            pltpu.make_async_copy(table_hbm.at[idx_all.at[pl.ds(o0, CHUNK)]],
                                  rowsa, gsem).wait()
            pltpu.make_async_copy(
                table_hbm.at[idx_all.at[pl.ds(o0 + BLK, CHUNK)]],
                rowsb, gsem).wait()

        fire(0, rowsa0, rowsb0, gsem0)

        def compute(rowsa, rowsb, outb):
            for grp in range(D // (L * GROUP)):
                wslices = []
                for j in range(GROUP):
                    dsl = pl.ds((grp * GROUP + j) * L, L)
                    wslices.append((w_v[0, dsl], w_v[1, dsl], dsl))

                @plsc.parallel_loop(0, CHUNK, unroll=1)
                def _(c):
                    for w0, w1, dsl in wslices:
                        r0 = rowsa[c, dsl]
                        r1 = rowsb[c, dsl]
                        outb[c, dsl] = jnp.maximum(r0 * w0 + r1 * w1, 0.0)

        def pair_body(i, carry):
            g0 = 2 * i
            g1 = g0 + 1
            base0 = (wid * n_chunks + g0) * CHUNK
            base1 = base0 + CHUNK
            fire(g1, rowsa1, rowsb1, gsem1)
            drain(g0, rowsa0, rowsb0, gsem0)

            @pl.when(i > 0)
            def _():
                pltpu.make_async_copy(
                    outb0, out_hbm.at[pl.ds(base0, CHUNK), 0], osem0).wait()

            compute(rowsa0, rowsb0, outb0)
            pltpu.async_copy(outb0, out_hbm.at[pl.ds(base0, CHUNK), 0], osem0)

            @pl.when(i < n_pairs - 1)
            def _():
                fire(g0 + 2, rowsa0, rowsb0, gsem0)

            drain(g1, rowsa1, rowsb1, gsem1)

            @pl.when(i > 0)
            def _():
                pltpu.make_async_copy(
                    outb1, out_hbm.at[pl.ds(base1, CHUNK), 0], osem1).wait()

            compute(rowsa1, rowsb1, outb1)
            pltpu.async_copy(outb1, out_hbm.at[pl.ds(base1, CHUNK), 0], osem1)
            return carry

        lax.fori_loop(0, n_pairs, pair_body, 0)
        pltpu.make_async_copy(outb0, out_hbm.at[pl.ds(0, CHUNK), 0], osem0).wait()
        pltpu.make_async_copy(outb1, out_hbm.at[pl.ds(0, CHUNK), 0], osem1).wait()

    return decoder_kernel


@jax.jit
def kernel(y, table, emb_scale, conv_w, conv_scale):
    B, ctx = y.shape
    V, D = table.shape
    w = (conv_w * jnp.exp(emb_scale + conv_scale)).astype(jnp.float32)
    w_t = w.T
    n_chunks = B // (NW * CHUNK)
    idx = (y.astype(jnp.int32)
            .reshape(B // BLK, BLK, ctx)
            .transpose(0, 2, 1)
            .reshape(-1))

    return _make_sc_kernel(B, D, n_chunks)(idx, table, w_t)
